# baseline (speedup 1.0000x reference)
import dataclasses
import functools

import jax
import jax.numpy as jnp
from jax import lax
from jax.experimental import pallas as pl
from jax.experimental.pallas import tpu as pltpu
from jax.experimental.pallas import tpu_sc as plsc

_BATCH = 200
_NUM_NODES = 50
_D = 128
_NG = 10
_GS = 5
_N1 = _BATCH * _NUM_NODES
_N2 = _BATCH * _NG
_E1 = 320000
_E2 = 64000
_NC = 2
_NS = 16
_NW = _NC * _NS
_K = 80

_mesh = plsc.VectorSubcoreMesh(core_axis_name="c", subcore_axis_name="s")
_sc_params = dataclasses.replace(pltpu.CompilerParams(),
                                 needs_layout_passes=False)
_sc_lin_params = dataclasses.replace(pltpu.CompilerParams(),
                                     use_tc_tiling_on_sc=False)

_HIGH = lax.Precision.HIGHEST


def _sc_degrees(dst1, dst2):
    epw1 = _E1 // _NW
    epw2 = _E2 // _NW
    cw = 2000

    @functools.partial(
        pl.kernel,
        out_type=(
            jax.ShapeDtypeStruct((_NW, _N1), jnp.float32),
            jax.ShapeDtypeStruct((_NW, _N2), jnp.float32),
        ),
        mesh=_mesh,
        scratch_types=[
            pltpu.VMEM((_N1,), jnp.float32),
            pltpu.VMEM((_N2,), jnp.float32),
            pltpu.VMEM((cw,), jnp.int32),
        ],
        compiler_params=_sc_params,
    )
    def k(d1_hbm, d2_hbm, o1_hbm, o2_hbm, h1, h2, ibuf):
        c = lax.axis_index("c")
        s = lax.axis_index("s")
        wid = c * _NS + s
        zero16 = jnp.zeros((16,), jnp.float32)
        one16 = jnp.ones((16,), jnp.float32)

        @pl.loop(0, _N1 // 16)
        def _(i):
            h1[pl.ds(i * 16, 16)] = zero16

        @pl.loop(0, _N2 // 16)
        def _(i):
            h2[pl.ds(i * 16, 16)] = zero16

        @pl.loop(0, epw1 // cw)
        def _(j):
            pltpu.sync_copy(d1_hbm.at[pl.ds(wid * epw1 + j * cw, cw)], ibuf)

            @pl.loop(0, cw // 16)
            def _(t):
                idx = ibuf[pl.ds(t * 16, 16)]
                plsc.addupdate_scatter(h1, [idx], one16)

        @pl.loop(0, epw2 // cw)
        def _(j):
            pltpu.sync_copy(d2_hbm.at[pl.ds(wid * epw2 + j * cw, cw)], ibuf)

            @pl.loop(0, cw // 16)
            def _(t):
                idx = ibuf[pl.ds(t * 16, 16)]
                plsc.addupdate_scatter(h2, [idx], one16)

        pltpu.sync_copy(h1, o1_hbm.at[wid])
        pltpu.sync_copy(h2, o2_hbm.at[wid])

    return k(dst1, dst2)


_DH = _D // 2


def _sc_edge_scatter(hs_split, src, dst, n_nodes, n_edges):
    epw = n_edges // _NS
    nsteps = epw // _K
    _T = 50
    _NB = 10
    _D5 = _NB // 2
    n_super = nsteps // _T
    assert n_super * _T == nsteps and _T % _NB == 0
    rpw = ((n_nodes // _NS) + 7) // 8 * 8
    n_pad = rpw * _NS
    last = n_nodes - (_NS - 1) * rpw

    @functools.partial(
        pl.kernel,
        out_type=jax.ShapeDtypeStruct((_NC, n_nodes, _DH), jnp.float32),
        mesh=_mesh,
        scratch_types=[
            pltpu.VMEM((_T * _K,), jnp.int32),
            pltpu.VMEM((2, _T, _K), jnp.int32),
            pltpu.VMEM((_K,), jnp.int32),
            pltpu.VMEM((_NB, _K, _DH), jnp.float32),
            pltpu.VMEM_SHARED((n_pad, _DH), jnp.float32),
            pltpu.SemaphoreType.DMA((_NB,)),
            pltpu.SemaphoreType.DMA((_NB,)),
        ],
        compiler_params=_sc_lin_params,
    )
    def k(hs_hbm, src_hbm, dst2_hbm, out_hbm, sidxb, didxb, zidx, rows, acc,
          gsem, ssem):
        c = lax.axis_index("c")
        s = lax.axis_index("s")
        zero16 = jnp.zeros((16,), jnp.float32)
        row_off = pl.multiple_of(s * rpw, 8)

        @pl.loop(0, _K)
        def _(i):
            for b in (0,) + tuple(range(_D5, _NB)):
                for j in range(_DH // 16):
                    rows[b, i, pl.ds(j * 16, 16)] = zero16

        @pl.loop(0, rpw // _K)
        def _(i):
            pltpu.sync_copy(rows.at[0], acc.at[pl.ds(row_off + i * _K, _K)])

        ztail = rpw % _K
        if ztail:
            pltpu.sync_copy(
                rows.at[0, pl.ds(0, ztail)],
                acc.at[pl.ds(row_off + (rpw // _K) * _K, ztail)])

        for j in range(_K // 16):
            zidx[pl.ds(j * 16, 16)] = lax.iota(jnp.int32, 16) + (j * 16)

        plsc.subcore_barrier()

        tbl = hs_hbm.at[c]

        def gather(j, b):
            pltpu.async_copy(
                tbl.at[sidxb.at[pl.ds(j * _K, _K)]], rows.at[b], gsem.at[b])

        def gather_wait(j, b):
            pltpu.make_async_copy(
                tbl.at[sidxb.at[pl.ds(j * _K, _K)]], rows.at[b],
                gsem.at[b]).wait()

        def scat(p, j, b):
            pltpu.async_copy(rows.at[b], acc.at[didxb.at[p, j]], ssem.at[b],
                             add=True)

        def scat_wait(p, j, b):
            jc = jnp.maximum(j, 0) if not isinstance(j, int) else max(j, 0)
            pltpu.make_async_copy(rows.at[b], acc.at[didxb.at[p, jc]],
                                  ssem.at[b]).wait()

        for b in range(_D5, _NB):
            pltpu.async_copy(rows.at[b], acc.at[zidx], ssem.at[b], add=True)

        @pl.loop(0, n_super)
        def _(sb):
            p = sb % 2
            e_off = s * epw + sb * (_T * _K)
            pltpu.sync_copy(src_hbm.at[pl.ds(e_off, _T * _K)], sidxb)
            pltpu.sync_copy(dst2_hbm.at[pl.ds(s * nsteps + sb * _T, _T)],
                            didxb.at[p])
            for b in range(_D5):
                gather(b, b)

            @pl.loop(0, _T // _NB - 1)
            def _(jj):
                for b in range(_NB):
                    j = jj * _NB + b
                    u = (b + _D5) % _NB
                    gather_wait(j, b)
                    scat(p, j, b)
                    scat_wait(p, j - _D5, u)
                    gather(j + _D5, u)

            for b in range(_NB):
                j = _T - _NB + b
                u = (b + _D5) % _NB
                gather_wait(j, b)
                scat(p, j, b)
                scat_wait(p, j - _D5, u)
                if b < _D5:
                    gather(j + _D5, u)

        for b in range(_D5, _NB):
            scat_wait((n_super - 1) % 2, _T - _NB + b, b)

        plsc.subcore_barrier()

        @pl.when(s < _NS - 1)
        def _():
            pltpu.sync_copy(acc.at[pl.ds(row_off, rpw)],
                            out_hbm.at[c, pl.ds(row_off, rpw)])

        @pl.when(s == _NS - 1)
        def _():
            pltpu.sync_copy(acc.at[pl.ds((_NS - 1) * rpw, last)],
                            out_hbm.at[c, pl.ds((_NS - 1) * rpw, last)])

    return k(hs_split, src, dst.reshape(n_edges // _K, _K))


def _deg_col(parts_ref, nw):
    ones = jnp.ones((nw, 1), jnp.float32)
    d = lax.dot_general(parts_ref[...], ones, (((0,), (0,)), ((), ())),
                        precision=_HIGH, preferred_element_type=jnp.float32)
    return lax.rsqrt(d + 1.0)


def _tc_prep(x, W1, deg1_parts, deg2_parts):
    def body(x_ref, w_ref, dp1_ref, dp2_ref, hs_ref, dis1_ref, dis2_ref):
        dis1 = _deg_col(dp1_ref, _NW)
        dis2 = _deg_col(dp2_ref, _NW)
        h = jnp.dot(x_ref[...], w_ref[...], precision=_HIGH,
                    preferred_element_type=jnp.float32)
        hs = h * dis1
        hs_ref[0] = hs[:, :_DH]
        hs_ref[1] = hs[:, _DH:]
        dis1_ref[...] = dis1
        dis2_ref[...] = dis2

    return pl.pallas_call(
        body,
        out_shape=[
            jax.ShapeDtypeStruct((_NC, _N1, _DH), jnp.float32),
            jax.ShapeDtypeStruct((_N1, 1), jnp.float32),
            jax.ShapeDtypeStruct((_N2, 1), jnp.float32),
        ],
    )(x, W1, deg1_parts, deg2_parts)


def _tc_mid(acc1_parts, hs1, dis1, b1, agg_W, agg_b, W2, dis2):
    def body(a_ref, hs_ref, dis1_ref, b1_ref, aw_ref, ab_ref, w2_ref, dis2_ref,
             hs2_ref):
        full = jnp.concatenate(
            [a_ref[0] + hs_ref[0], a_ref[1] + hs_ref[1]], axis=1)
        h1 = full * dis1_ref[...] + b1_ref[...]
        h1r = h1.reshape(_BATCH, _NUM_NODES, _D)
        dis2r = dis2_ref[...].reshape(_BATCH, _NG, 1)
        for g in range(_NG):
            z = ab_ref[g][None, :] + jnp.zeros((_BATCH, _D), jnp.float32)
            for kk in range(_GS):
                z = z + jnp.dot(h1r[:, g * _GS + kk, :], aw_ref[g, kk],
                                precision=_HIGH,
                                preferred_element_type=jnp.float32)
            g2 = jnp.dot(z, w2_ref[...], precision=_HIGH,
                         preferred_element_type=jnp.float32)
            hs2 = g2 * dis2r[:, g, :]
            hs2_ref[0, :, g, :] = hs2[:, :_DH]
            hs2_ref[1, :, g, :] = hs2[:, _DH:]

    return pl.pallas_call(
        body,
        out_shape=jax.ShapeDtypeStruct((_NC, _BATCH, _NG, _DH), jnp.float32),
    )(acc1_parts, hs1, dis1, b1, agg_W, agg_b, W2, dis2)


def _tc_final(acc2_parts, hs2, dis2, b2):
    def body(a_ref, hs_ref, dis_ref, b_ref, o_ref):
        full = jnp.concatenate(
            [a_ref[0] + hs_ref[0], a_ref[1] + hs_ref[1]], axis=1)
        o_ref[...] = full * dis_ref[...] + b_ref[...]

    return pl.pallas_call(
        body,
        out_shape=jax.ShapeDtypeStruct((_N2, _D), jnp.float32),
    )(acc2_parts, hs2, dis2, b2)


def kernel(x, A1, A2, W1, b1, agg_W, agg_b, W2, b2):
    src1, dst1 = A1[0], A1[1]
    src2, dst2 = A2[0], A2[1]

    deg1_parts, deg2_parts = _sc_degrees(dst1, dst2)

    hs1s, dis1, dis2 = _tc_prep(x, W1, deg1_parts, deg2_parts)

    acc1_parts = _sc_edge_scatter(hs1s, src1, dst1, _N1, _E1)

    hs2s = _tc_mid(
        acc1_parts, hs1s, dis1, b1,
        agg_W.reshape(_NG, _GS, _D, _D), agg_b, W2, dis2,
    ).reshape(_NC, _N2, _DH)

    acc2_parts = _sc_edge_scatter(hs2s, src2, dst2, _N2, _E2)

    out = _tc_final(acc2_parts, hs2s, dis2, b2)
    return out


# speedup vs baseline: 1.0167x; 1.0167x over previous
import dataclasses
import functools

import jax
import jax.numpy as jnp
from jax import lax
from jax.experimental import pallas as pl
from jax.experimental.pallas import tpu as pltpu
from jax.experimental.pallas import tpu_sc as plsc

_BATCH = 200
_NUM_NODES = 50
_D = 128
_NG = 10
_GS = 5
_N1 = _BATCH * _NUM_NODES
_N2 = _BATCH * _NG
_E1 = 320000
_E2 = 64000
_NC = 2
_NS = 16
_NW = _NC * _NS
_K = 80

_mesh = plsc.VectorSubcoreMesh(core_axis_name="c", subcore_axis_name="s")
_sc_params = dataclasses.replace(pltpu.CompilerParams(),
                                 needs_layout_passes=False)
_sc_lin_params = dataclasses.replace(pltpu.CompilerParams(),
                                     use_tc_tiling_on_sc=False)

_HIGH = lax.Precision.HIGHEST


def _sc_degrees(dst1, dst2):
    epw1 = _E1 // _NW
    epw2 = _E2 // _NW
    cw = 2000

    @functools.partial(
        pl.kernel,
        out_type=(
            jax.ShapeDtypeStruct((_NW, _N1), jnp.float32),
            jax.ShapeDtypeStruct((_NW, _N2), jnp.float32),
        ),
        mesh=_mesh,
        scratch_types=[
            pltpu.VMEM((_N1,), jnp.float32),
            pltpu.VMEM((_N2,), jnp.float32),
            pltpu.VMEM((cw,), jnp.int32),
        ],
        compiler_params=_sc_params,
    )
    def k(d1_hbm, d2_hbm, o1_hbm, o2_hbm, h1, h2, ibuf):
        c = lax.axis_index("c")
        s = lax.axis_index("s")
        wid = c * _NS + s
        zero16 = jnp.zeros((16,), jnp.float32)
        one16 = jnp.ones((16,), jnp.float32)

        @pl.loop(0, _N1 // 16)
        def _(i):
            h1[pl.ds(i * 16, 16)] = zero16

        @pl.loop(0, _N2 // 16)
        def _(i):
            h2[pl.ds(i * 16, 16)] = zero16

        @pl.loop(0, epw1 // cw)
        def _(j):
            pltpu.sync_copy(d1_hbm.at[pl.ds(wid * epw1 + j * cw, cw)], ibuf)

            @pl.loop(0, cw // 16)
            def _(t):
                idx = ibuf[pl.ds(t * 16, 16)]
                plsc.addupdate_scatter(h1, [idx], one16)

        @pl.loop(0, epw2 // cw)
        def _(j):
            pltpu.sync_copy(d2_hbm.at[pl.ds(wid * epw2 + j * cw, cw)], ibuf)

            @pl.loop(0, cw // 16)
            def _(t):
                idx = ibuf[pl.ds(t * 16, 16)]
                plsc.addupdate_scatter(h2, [idx], one16)

        pltpu.sync_copy(h1, o1_hbm.at[wid])
        pltpu.sync_copy(h2, o2_hbm.at[wid])

    return k(dst1, dst2)


_DH = _D // 2


def _sc_edge_scatter(hs_split, src, dst, n_nodes, n_edges,
                     finalize=None):
    epw = n_edges // _NS
    nsteps = epw // _K
    _T = 50
    _NB = 10
    n_super = nsteps // _T
    assert n_super * _T == nsteps and _T % _NB == 0
    rpw = ((n_nodes // _NS) + 7) // 8 * 8
    n_pad = rpw * _NS
    last = n_nodes - (_NS - 1) * rpw

    if finalize is None:
        out_type = jax.ShapeDtypeStruct((_NC, n_nodes, _DH), jnp.float32)
        extra_in = ()
    else:
        out_type = jax.ShapeDtypeStruct((n_nodes, _D), jnp.float32)
        extra_in = finalize

    @functools.partial(
        pl.kernel,
        out_type=out_type,
        mesh=_mesh,
        scratch_types=[
            pltpu.VMEM((_T * _K,), jnp.int32),
            pltpu.VMEM((_T, _K), jnp.int32),
            pltpu.VMEM((_NB, _K, _DH), jnp.float32),
            pltpu.VMEM((_K, 16), jnp.float32),
            pltpu.VMEM((1, _DH), jnp.float32),
            pltpu.VMEM_SHARED((n_pad, _DH), jnp.float32),
            pltpu.SemaphoreType.DMA((_NB,)),
        ],
        compiler_params=_sc_lin_params,
    )
    def k(hs_hbm, src_hbm, dst2_hbm, *rest):
        if finalize is None:
            out_hbm, sidxb, didxb, rows, disb, bb, acc, gsem = rest
            dis_hbm = b_hbm = None
        else:
            dis_hbm, b_hbm, out_hbm, sidxb, didxb, rows, disb, bb, acc, \
                gsem = rest
        c = lax.axis_index("c")
        s = lax.axis_index("s")
        zero16 = jnp.zeros((16,), jnp.float32)
        row_off = pl.multiple_of(s * rpw, 8)

        @pl.loop(0, _K)
        def _(i):
            for j in range(_DH // 16):
                rows[0, i, pl.ds(j * 16, 16)] = zero16

        @pl.loop(0, rpw // _K)
        def _(i):
            pltpu.sync_copy(rows.at[0], acc.at[pl.ds(row_off + i * _K, _K)])

        ztail = rpw % _K
        if ztail:
            pltpu.sync_copy(
                rows.at[0, pl.ds(0, ztail)],
                acc.at[pl.ds(row_off + (rpw // _K) * _K, ztail)])

        plsc.subcore_barrier()

        tbl = hs_hbm.at[c]

        def gather(j, b):
            pltpu.async_copy(
                tbl.at[sidxb.at[pl.ds(j * _K, _K)]], rows.at[b], gsem.at[b])

        def gather_wait(j, b):
            pltpu.make_async_copy(
                tbl.at[sidxb.at[pl.ds(j * _K, _K)]], rows.at[b],
                gsem.at[b]).wait()

        @pl.loop(0, n_super)
        def _(sb):
            e_off = s * epw + sb * (_T * _K)
            pltpu.sync_copy(src_hbm.at[pl.ds(e_off, _T * _K)], sidxb)
            pltpu.sync_copy(dst2_hbm.at[pl.ds(s * nsteps + sb * _T, _T)],
                            didxb)
            for b in range(_NB):
                gather(b, b)

            @pl.loop(0, _T // _NB - 1)
            def _(jj):
                for b in range(_NB):
                    j = jj * _NB + b
                    gather_wait(j, b)
                    pltpu.sync_copy(rows.at[b], acc.at[didxb.at[j]], add=True)
                    gather(j + _NB, b)

            for b in range(_NB):
                j = _T - _NB + b
                gather_wait(j, b)
                pltpu.sync_copy(rows.at[b], acc.at[didxb.at[j]], add=True)

        plsc.subcore_barrier()

        if finalize is None:
            @pl.when(s < _NS - 1)
            def _():
                pltpu.sync_copy(acc.at[pl.ds(row_off, rpw)],
                                out_hbm.at[c, pl.ds(row_off, rpw)])

            @pl.when(s == _NS - 1)
            def _():
                pltpu.sync_copy(acc.at[pl.ds((_NS - 1) * rpw, last)],
                                out_hbm.at[c, pl.ds((_NS - 1) * rpw, last)])
        else:
            pltpu.sync_copy(b_hbm.at[pl.ds(c, 1)], bb)
            bvecs = [bb[0, pl.ds(j * 16, 16)] for j in range(_DH // 16)]

            def fin_group(go, gn):
                gbase = row_off + go
                pltpu.sync_copy(acc.at[pl.ds(gbase, gn)],
                                rows.at[0, pl.ds(0, gn)])
                pltpu.sync_copy(tbl.at[pl.ds(gbase, gn)],
                                rows.at[1, pl.ds(0, gn)])
                pltpu.sync_copy(dis_hbm.at[pl.ds(gbase, gn)],
                                disb.at[pl.ds(0, gn)])

                @pl.loop(0, gn)
                def _(r):
                    dv = disb[r]
                    for j in range(_DH // 16):
                        sl = pl.ds(j * 16, 16)
                        rows[2, r, sl] = ((rows[0, r, sl] + rows[1, r, sl])
                                          * dv + bvecs[j])

                pltpu.sync_copy(
                    rows.at[2, pl.ds(0, gn)],
                    out_hbm.at[pl.ds(gbase, gn), pl.ds(c * _DH, _DH)])

            assert last <= _K

            @pl.when(s < _NS - 1)
            def _():
                for go, gn in ([(0, _K), (_K, rpw - _K)] if rpw > _K
                               else [(0, rpw)]):
                    fin_group(go, gn)

            @pl.when(s == _NS - 1)
            def _():
                fin_group(0, last)

    return k(hs_split, src, dst.reshape(n_edges // _K, _K), *extra_in)


def _deg_col(parts_ref, nw):
    ones = jnp.ones((nw, 1), jnp.float32)
    d = lax.dot_general(parts_ref[...], ones, (((0,), (0,)), ((), ())),
                        precision=_HIGH, preferred_element_type=jnp.float32)
    return lax.rsqrt(d + 1.0)


def _tc_prep(x, W1, deg1_parts, deg2_parts):
    def body(x_ref, w_ref, dp1_ref, dp2_ref, hs_ref, dis1_ref, dis2_ref):
        dis1 = _deg_col(dp1_ref, _NW)
        dis2 = _deg_col(dp2_ref, _NW)
        h = jnp.dot(x_ref[...], w_ref[...], precision=_HIGH,
                    preferred_element_type=jnp.float32)
        hs = h * dis1
        hs_ref[0] = hs[:, :_DH]
        hs_ref[1] = hs[:, _DH:]
        dis1_ref[...] = dis1
        dis2_ref[...] = dis2

    return pl.pallas_call(
        body,
        out_shape=[
            jax.ShapeDtypeStruct((_NC, _N1, _DH), jnp.float32),
            jax.ShapeDtypeStruct((_N1, 1), jnp.float32),
            jax.ShapeDtypeStruct((_N2, 1), jnp.float32),
        ],
    )(x, W1, deg1_parts, deg2_parts)


def _tc_mid(acc1_parts, hs1, dis1, b1, agg_W, agg_b, W2, dis2):
    def body(a_ref, hs_ref, dis1_ref, b1_ref, aw_ref, ab_ref, w2_ref, dis2_ref,
             hs2_ref, dis2b_ref):
        full = jnp.concatenate(
            [a_ref[0] + hs_ref[0], a_ref[1] + hs_ref[1]], axis=1)
        h1 = full * dis1_ref[...] + b1_ref[...]
        h1r = h1.reshape(_BATCH, _NUM_NODES, _D)
        dis2r = dis2_ref[...].reshape(_BATCH, _NG, 1)
        for g in range(_NG):
            z = ab_ref[g][None, :] + jnp.zeros((_BATCH, _D), jnp.float32)
            for kk in range(_GS):
                z = z + jnp.dot(h1r[:, g * _GS + kk, :], aw_ref[g, kk],
                                precision=_HIGH,
                                preferred_element_type=jnp.float32)
            g2 = jnp.dot(z, w2_ref[...], precision=_HIGH,
                         preferred_element_type=jnp.float32)
            hs2 = g2 * dis2r[:, g, :]
            hs2_ref[0, :, g, :] = hs2[:, :_DH]
            hs2_ref[1, :, g, :] = hs2[:, _DH:]
            dis2b_ref[:, g, :] = jnp.broadcast_to(dis2r[:, g, :], (_BATCH, 16))

    return pl.pallas_call(
        body,
        out_shape=[
            jax.ShapeDtypeStruct((_NC, _BATCH, _NG, _DH), jnp.float32),
            jax.ShapeDtypeStruct((_BATCH, _NG, 16), jnp.float32),
        ],
    )(acc1_parts, hs1, dis1, b1, agg_W, agg_b, W2, dis2)


def kernel(x, A1, A2, W1, b1, agg_W, agg_b, W2, b2):
    src1, dst1 = A1[0], A1[1]
    src2, dst2 = A2[0], A2[1]

    deg1_parts, deg2_parts = _sc_degrees(dst1, dst2)

    hs1s, dis1, dis2 = _tc_prep(x, W1, deg1_parts, deg2_parts)

    acc1_parts = _sc_edge_scatter(hs1s, src1, dst1, _N1, _E1)

    hs2s, dis2b = _tc_mid(
        acc1_parts, hs1s, dis1, b1,
        agg_W.reshape(_NG, _GS, _D, _D), agg_b, W2, dis2,
    )
    hs2s = hs2s.reshape(_NC, _N2, _DH)

    out = _sc_edge_scatter(
        hs2s, src2, dst2, _N2, _E2,
        finalize=(dis2b.reshape(_N2, 16), b2.reshape(_NC, _DH)))
    return out


# speedup vs baseline: 1.0261x; 1.0092x over previous
import dataclasses
import functools

import jax
import jax.numpy as jnp
from jax import lax
from jax.experimental import pallas as pl
from jax.experimental.pallas import tpu as pltpu
from jax.experimental.pallas import tpu_sc as plsc

_BATCH = 200
_NUM_NODES = 50
_D = 128
_NG = 10
_GS = 5
_N1 = _BATCH * _NUM_NODES
_N2 = _BATCH * _NG
_E1 = 320000
_E2 = 64000
_NC = 2
_NS = 16
_NW = _NC * _NS
_K = 80

_mesh = plsc.VectorSubcoreMesh(core_axis_name="c", subcore_axis_name="s")
_sc_params = dataclasses.replace(pltpu.CompilerParams(),
                                 needs_layout_passes=False)
_sc_lin_params = dataclasses.replace(pltpu.CompilerParams(),
                                     use_tc_tiling_on_sc=False)

_HIGH = lax.Precision.HIGHEST


def _sc_degrees(dst1, dst2):
    epw1 = _E1 // _NW
    epw2 = _E2 // _NW
    cw = 2000

    @functools.partial(
        pl.kernel,
        out_type=(
            jax.ShapeDtypeStruct((_NW, _N1), jnp.float32),
            jax.ShapeDtypeStruct((_NW, _N2), jnp.float32),
        ),
        mesh=_mesh,
        scratch_types=[
            pltpu.VMEM((_N1,), jnp.float32),
            pltpu.VMEM((_N2,), jnp.float32),
            pltpu.VMEM((cw,), jnp.int32),
        ],
        compiler_params=_sc_params,
    )
    def k(d1_hbm, d2_hbm, o1_hbm, o2_hbm, h1, h2, ibuf):
        c = lax.axis_index("c")
        s = lax.axis_index("s")
        wid = c * _NS + s
        zero16 = jnp.zeros((16,), jnp.float32)
        one16 = jnp.ones((16,), jnp.float32)

        @pl.loop(0, _N1 // 16)
        def _(i):
            h1[pl.ds(i * 16, 16)] = zero16

        @pl.loop(0, _N2 // 16)
        def _(i):
            h2[pl.ds(i * 16, 16)] = zero16

        @pl.loop(0, epw1 // cw)
        def _(j):
            pltpu.sync_copy(d1_hbm.at[pl.ds(wid * epw1 + j * cw, cw)], ibuf)

            @pl.loop(0, cw // 16)
            def _(t):
                idx = ibuf[pl.ds(t * 16, 16)]
                plsc.addupdate_scatter(h1, [idx], one16)

        @pl.loop(0, epw2 // cw)
        def _(j):
            pltpu.sync_copy(d2_hbm.at[pl.ds(wid * epw2 + j * cw, cw)], ibuf)

            @pl.loop(0, cw // 16)
            def _(t):
                idx = ibuf[pl.ds(t * 16, 16)]
                plsc.addupdate_scatter(h2, [idx], one16)

        pltpu.sync_copy(h1, o1_hbm.at[wid])
        pltpu.sync_copy(h2, o2_hbm.at[wid])

    return k(dst1, dst2)


_DH = _D // 2


def _sc_edge_scatter(hs_split, src, dst, n_nodes, n_edges,
                     finalize=None):
    epw = n_edges // _NS
    nsteps = epw // _K
    _T = 50
    _NB = 10
    n_super = nsteps // _T
    assert n_super * _T == nsteps and _T % _NB == 0
    rpw = ((n_nodes // _NS) + 7) // 8 * 8
    n_pad = rpw * _NS
    last = n_nodes - (_NS - 1) * rpw

    if finalize is None:
        out_type = jax.ShapeDtypeStruct((_NC, n_nodes, _DH), jnp.float32)
        extra_in = ()
    else:
        out_type = jax.ShapeDtypeStruct((n_nodes, _D), jnp.float32)
        extra_in = finalize

    @functools.partial(
        pl.kernel,
        out_type=out_type,
        mesh=_mesh,
        scratch_types=[
            pltpu.VMEM((_T * _K,), jnp.int32),
            pltpu.VMEM((_T, _K), jnp.int32),
            pltpu.VMEM((_NB, _K, _DH), jnp.float32),
            pltpu.VMEM((_K, 16), jnp.float32),
            pltpu.VMEM((1, _DH), jnp.float32),
            pltpu.VMEM_SHARED((n_pad, _DH), jnp.float32),
            pltpu.SemaphoreType.DMA((_NB,)),
        ],
        compiler_params=_sc_lin_params,
    )
    def k(hs_hbm, src_hbm, dst2_hbm, *rest):
        if finalize is None:
            out_hbm, sidxb, didxb, rows, disb, bb, acc, gsem = rest
            dis_hbm = b_hbm = None
        else:
            dis_hbm, b_hbm, out_hbm, sidxb, didxb, rows, disb, bb, acc, \
                gsem = rest
        c = lax.axis_index("c")
        s = lax.axis_index("s")
        zero16 = jnp.zeros((16,), jnp.float32)
        row_off = pl.multiple_of(s * rpw, 8)

        @pl.loop(0, _K)
        def _(i):
            for j in range(_DH // 16):
                rows[0, i, pl.ds(j * 16, 16)] = zero16

        @pl.loop(0, rpw // _K)
        def _(i):
            pltpu.sync_copy(rows.at[0], acc.at[pl.ds(row_off + i * _K, _K)])

        ztail = rpw % _K
        if ztail:
            pltpu.sync_copy(
                rows.at[0, pl.ds(0, ztail)],
                acc.at[pl.ds(row_off + (rpw // _K) * _K, ztail)])

        plsc.subcore_barrier()

        tbl = hs_hbm.at[c]

        def gather(j, b):
            pltpu.async_copy(
                tbl.at[sidxb.at[pl.ds(j * _K, _K)]], rows.at[b], gsem.at[b])

        def gather_wait(j, b):
            pltpu.make_async_copy(
                tbl.at[sidxb.at[pl.ds(j * _K, _K)]], rows.at[b],
                gsem.at[b]).wait()

        @pl.loop(0, n_super)
        def _(sb):
            e_off = s * epw + sb * (_T * _K)
            pltpu.sync_copy(src_hbm.at[pl.ds(e_off, _T * _K)], sidxb)
            pltpu.sync_copy(dst2_hbm.at[pl.ds(s * nsteps + sb * _T, _T)],
                            didxb)
            for b in range(_NB):
                gather(b, b)

            @pl.loop(0, _T // _NB - 1)
            def _(jj):
                for b in range(_NB):
                    j = jj * _NB + b
                    gather_wait(j, b)
                    pltpu.sync_copy(rows.at[b], acc.at[didxb.at[j]], add=True)
                    gather(j + _NB, b)

            for b in range(_NB):
                j = _T - _NB + b
                gather_wait(j, b)
                pltpu.sync_copy(rows.at[b], acc.at[didxb.at[j]], add=True)

        plsc.subcore_barrier()

        if finalize is None:
            @pl.when(s < _NS - 1)
            def _():
                pltpu.sync_copy(acc.at[pl.ds(row_off, rpw)],
                                out_hbm.at[c, pl.ds(row_off, rpw)])

            @pl.when(s == _NS - 1)
            def _():
                pltpu.sync_copy(acc.at[pl.ds((_NS - 1) * rpw, last)],
                                out_hbm.at[c, pl.ds((_NS - 1) * rpw, last)])
        else:
            pltpu.sync_copy(b_hbm.at[pl.ds(c, 1)], bb)
            bvecs = [bb[0, pl.ds(j * 16, 16)] for j in range(_DH // 16)]

            def fin_group(go, gn):
                gbase = row_off + go
                pltpu.sync_copy(acc.at[pl.ds(gbase, gn)],
                                rows.at[0, pl.ds(0, gn)])
                pltpu.sync_copy(tbl.at[pl.ds(gbase, gn)],
                                rows.at[1, pl.ds(0, gn)])
                pltpu.sync_copy(dis_hbm.at[pl.ds(gbase, gn)],
                                disb.at[pl.ds(0, gn)])

                @pl.loop(0, gn)
                def _(r):
                    dv = disb[r]
                    for j in range(_DH // 16):
                        sl = pl.ds(j * 16, 16)
                        rows[2, r, sl] = ((rows[0, r, sl] + rows[1, r, sl])
                                          * dv + bvecs[j])

                pltpu.sync_copy(
                    rows.at[2, pl.ds(0, gn)],
                    out_hbm.at[pl.ds(gbase, gn), pl.ds(c * _DH, _DH)])

            assert last <= _K

            @pl.when(s < _NS - 1)
            def _():
                for go, gn in ([(0, _K), (_K, rpw - _K)] if rpw > _K
                               else [(0, rpw)]):
                    fin_group(go, gn)

            @pl.when(s == _NS - 1)
            def _():
                fin_group(0, last)

    return k(hs_split, src, dst.reshape(n_edges // _K, _K), *extra_in)


def _deg_col(parts_ref, nw):
    ones = jnp.ones((nw, 1), jnp.float32)
    d = lax.dot_general(parts_ref[...], ones, (((0,), (0,)), ((), ())),
                        precision=_HIGH, preferred_element_type=jnp.float32)
    return lax.rsqrt(d + 1.0)


def _tc_matmul(x, W1):
    def body(x_ref, w_ref, h_ref):
        h_ref[...] = jnp.dot(x_ref[...], w_ref[...], precision=_HIGH,
                             preferred_element_type=jnp.float32)

    return pl.pallas_call(
        body,
        out_shape=jax.ShapeDtypeStruct((_N1, _D), jnp.float32),
    )(x, W1)


def _tc_scale(h, deg1_parts, deg2_parts):
    def body(h_ref, dp1_ref, dp2_ref, hs_ref, dis1_ref, dis2_ref):
        dis1 = _deg_col(dp1_ref, _NW)
        dis2 = _deg_col(dp2_ref, _NW)
        hs = h_ref[...] * dis1
        hs_ref[0] = hs[:, :_DH]
        hs_ref[1] = hs[:, _DH:]
        dis1_ref[...] = dis1
        dis2_ref[...] = dis2

    return pl.pallas_call(
        body,
        out_shape=[
            jax.ShapeDtypeStruct((_NC, _N1, _DH), jnp.float32),
            jax.ShapeDtypeStruct((_N1, 1), jnp.float32),
            jax.ShapeDtypeStruct((_N2, 1), jnp.float32),
        ],
    )(h, deg1_parts, deg2_parts)


def _tc_mid(acc1_parts, hs1, dis1, b1, agg_W, agg_b, W2, dis2):
    def body(a_ref, hs_ref, dis1_ref, b1_ref, aw_ref, ab_ref, w2_ref, dis2_ref,
             hs2_ref, dis2b_ref):
        full = jnp.concatenate(
            [a_ref[0] + hs_ref[0], a_ref[1] + hs_ref[1]], axis=1)
        h1 = full * dis1_ref[...] + b1_ref[...]
        h1r = h1.reshape(_BATCH, _NUM_NODES, _D)
        dis2r = dis2_ref[...].reshape(_BATCH, _NG, 1)
        for g in range(_NG):
            z = ab_ref[g][None, :] + jnp.zeros((_BATCH, _D), jnp.float32)
            for kk in range(_GS):
                z = z + jnp.dot(h1r[:, g * _GS + kk, :], aw_ref[g, kk],
                                precision=_HIGH,
                                preferred_element_type=jnp.float32)
            g2 = jnp.dot(z, w2_ref[...], precision=_HIGH,
                         preferred_element_type=jnp.float32)
            hs2 = g2 * dis2r[:, g, :]
            hs2_ref[0, :, g, :] = hs2[:, :_DH]
            hs2_ref[1, :, g, :] = hs2[:, _DH:]
            dis2b_ref[:, g, :] = jnp.broadcast_to(dis2r[:, g, :], (_BATCH, 16))

    return pl.pallas_call(
        body,
        out_shape=[
            jax.ShapeDtypeStruct((_NC, _BATCH, _NG, _DH), jnp.float32),
            jax.ShapeDtypeStruct((_BATCH, _NG, 16), jnp.float32),
        ],
    )(acc1_parts, hs1, dis1, b1, agg_W, agg_b, W2, dis2)


def kernel(x, A1, A2, W1, b1, agg_W, agg_b, W2, b2):
    src1, dst1 = A1[0], A1[1]
    src2, dst2 = A2[0], A2[1]

    deg1_parts, deg2_parts = _sc_degrees(dst1, dst2)
    h_raw = _tc_matmul(x, W1)

    hs1s, dis1, dis2 = _tc_scale(h_raw, deg1_parts, deg2_parts)

    acc1_parts = _sc_edge_scatter(hs1s, src1, dst1, _N1, _E1)

    hs2s, dis2b = _tc_mid(
        acc1_parts, hs1s, dis1, b1,
        agg_W.reshape(_NG, _GS, _D, _D), agg_b, W2, dis2,
    )
    hs2s = hs2s.reshape(_NC, _N2, _DH)

    out = _sc_edge_scatter(
        hs2s, src2, dst2, _N2, _E2,
        finalize=(dis2b.reshape(_N2, 16), b2.reshape(_NC, _DH)))
    return out


# speedup vs baseline: 1.0267x; 1.0006x over previous
import dataclasses
import functools

import jax
import jax.numpy as jnp
from jax import lax
from jax.experimental import pallas as pl
from jax.experimental.pallas import tpu as pltpu
from jax.experimental.pallas import tpu_sc as plsc

_BATCH = 200
_NUM_NODES = 50
_D = 128
_NG = 10
_GS = 5
_N1 = _BATCH * _NUM_NODES
_N2 = _BATCH * _NG
_E1 = 320000
_E2 = 64000
_NC = 2
_NS = 16
_NW = _NC * _NS
_K = 80

_mesh = plsc.VectorSubcoreMesh(core_axis_name="c", subcore_axis_name="s")
_sc_params = dataclasses.replace(pltpu.CompilerParams(),
                                 needs_layout_passes=False)
_sc_lin_params = dataclasses.replace(pltpu.CompilerParams(),
                                     use_tc_tiling_on_sc=False)

_HIGH = lax.Precision.HIGHEST


def _sc_degrees(dst1, dst2):
    epw1 = _E1 // _NW
    epw2 = _E2 // _NW
    cw = 2000

    @functools.partial(
        pl.kernel,
        out_type=(
            jax.ShapeDtypeStruct((_NW, _N1), jnp.float32),
            jax.ShapeDtypeStruct((_NW, _N2), jnp.float32),
        ),
        mesh=_mesh,
        scratch_types=[
            pltpu.VMEM((_N1,), jnp.float32),
            pltpu.VMEM((_N2,), jnp.float32),
            pltpu.VMEM((cw,), jnp.int32),
        ],
        compiler_params=_sc_params,
    )
    def k(d1_hbm, d2_hbm, o1_hbm, o2_hbm, h1, h2, ibuf):
        c = lax.axis_index("c")
        s = lax.axis_index("s")
        wid = c * _NS + s
        zero16 = jnp.zeros((16,), jnp.float32)
        one16 = jnp.ones((16,), jnp.float32)

        @pl.loop(0, _N1 // 16)
        def _(i):
            h1[pl.ds(i * 16, 16)] = zero16

        @pl.loop(0, _N2 // 16)
        def _(i):
            h2[pl.ds(i * 16, 16)] = zero16

        @pl.loop(0, epw1 // cw)
        def _(j):
            pltpu.sync_copy(d1_hbm.at[pl.ds(wid * epw1 + j * cw, cw)], ibuf)

            @pl.loop(0, cw // 16)
            def _(t):
                idx = ibuf[pl.ds(t * 16, 16)]
                plsc.addupdate_scatter(h1, [idx], one16)

        @pl.loop(0, epw2 // cw)
        def _(j):
            pltpu.sync_copy(d2_hbm.at[pl.ds(wid * epw2 + j * cw, cw)], ibuf)

            @pl.loop(0, cw // 16)
            def _(t):
                idx = ibuf[pl.ds(t * 16, 16)]
                plsc.addupdate_scatter(h2, [idx], one16)

        pltpu.sync_copy(h1, o1_hbm.at[wid])
        pltpu.sync_copy(h2, o2_hbm.at[wid])

    return k(dst1, dst2)


_DH = _D // 2


def _sc_edge_scatter(hs_split, src, dst, n_nodes, n_edges,
                     finalize=None):
    epw = n_edges // _NS
    nsteps = epw // _K
    _T = 50
    _NB = 10
    n_super = nsteps // _T
    assert n_super * _T == nsteps and _T % _NB == 0
    rpw = ((n_nodes // _NS) + 7) // 8 * 8
    n_pad = rpw * _NS
    last = n_nodes - (_NS - 1) * rpw

    if finalize is None:
        out_type = jax.ShapeDtypeStruct((_NC, n_nodes, _DH), jnp.float32)
        extra_in = ()
    else:
        out_type = jax.ShapeDtypeStruct((n_nodes, _D), jnp.float32)
        extra_in = finalize

    @functools.partial(
        pl.kernel,
        out_type=out_type,
        mesh=_mesh,
        scratch_types=[
            pltpu.VMEM((_T * _K,), jnp.int32),
            pltpu.VMEM((_T * _K,), jnp.int32),
            pltpu.VMEM((_NB, _K, _DH), jnp.float32),
            pltpu.VMEM((_K, 16), jnp.float32),
            pltpu.VMEM((1, _DH), jnp.float32),
            pltpu.VMEM_SHARED((n_pad, _DH), jnp.float32),
            pltpu.SemaphoreType.DMA((_NB,)),
        ],
        compiler_params=_sc_lin_params,
    )
    def k(hs_hbm, src_hbm, dst_hbm, *rest):
        if finalize is None:
            out_hbm, sidxb, didxb, rows, disb, bb, acc, gsem = rest
            dis_hbm = b_hbm = None
        else:
            dis_hbm, b_hbm, out_hbm, sidxb, didxb, rows, disb, bb, acc, \
                gsem = rest
        c = lax.axis_index("c")
        s = lax.axis_index("s")
        zero16 = jnp.zeros((16,), jnp.float32)
        row_off = pl.multiple_of(s * rpw, 8)

        @pl.loop(0, _K)
        def _(i):
            for j in range(_DH // 16):
                rows[0, i, pl.ds(j * 16, 16)] = zero16

        @pl.loop(0, rpw // _K)
        def _(i):
            pltpu.sync_copy(rows.at[0], acc.at[pl.ds(row_off + i * _K, _K)])

        ztail = rpw % _K
        if ztail:
            pltpu.sync_copy(
                rows.at[0, pl.ds(0, ztail)],
                acc.at[pl.ds(row_off + (rpw // _K) * _K, ztail)])

        plsc.subcore_barrier()

        tbl = hs_hbm.at[c]

        def gather(j, b):
            pltpu.async_copy(
                tbl.at[sidxb.at[pl.ds(j * _K, _K)]], rows.at[b], gsem.at[b])

        def gather_wait(j, b):
            pltpu.make_async_copy(
                tbl.at[sidxb.at[pl.ds(j * _K, _K)]], rows.at[b],
                gsem.at[b]).wait()

        @pl.loop(0, n_super)
        def _(sb):
            e_off = s * epw + sb * (_T * _K)
            pltpu.sync_copy(src_hbm.at[pl.ds(e_off, _T * _K)], sidxb)
            pltpu.sync_copy(dst_hbm.at[pl.ds(e_off, _T * _K)], didxb)
            for b in range(_NB):
                gather(b, b)

            @pl.loop(0, _T // _NB - 1)
            def _(jj):
                for b in range(_NB):
                    j = jj * _NB + b
                    gather_wait(j, b)
                    pltpu.sync_copy(rows.at[b],
                                    acc.at[didxb.at[pl.ds(j * _K, _K)]],
                                    add=True)
                    gather(j + _NB, b)

            for b in range(_NB):
                j = _T - _NB + b
                gather_wait(j, b)
                pltpu.sync_copy(rows.at[b],
                                acc.at[didxb.at[pl.ds(j * _K, _K)]],
                                add=True)

        plsc.subcore_barrier()

        if finalize is None:
            @pl.when(s < _NS - 1)
            def _():
                pltpu.sync_copy(acc.at[pl.ds(row_off, rpw)],
                                out_hbm.at[c, pl.ds(row_off, rpw)])

            @pl.when(s == _NS - 1)
            def _():
                pltpu.sync_copy(acc.at[pl.ds((_NS - 1) * rpw, last)],
                                out_hbm.at[c, pl.ds((_NS - 1) * rpw, last)])
        else:
            pltpu.sync_copy(b_hbm.at[pl.ds(c, 1)], bb)
            bvecs = [bb[0, pl.ds(j * 16, 16)] for j in range(_DH // 16)]

            def fin_group(go, gn):
                gbase = row_off + go
                pltpu.sync_copy(acc.at[pl.ds(gbase, gn)],
                                rows.at[0, pl.ds(0, gn)])
                pltpu.sync_copy(tbl.at[pl.ds(gbase, gn)],
                                rows.at[1, pl.ds(0, gn)])
                pltpu.sync_copy(dis_hbm.at[pl.ds(gbase, gn)],
                                disb.at[pl.ds(0, gn)])

                @pl.loop(0, gn)
                def _(r):
                    dv = disb[r]
                    for j in range(_DH // 16):
                        sl = pl.ds(j * 16, 16)
                        rows[2, r, sl] = ((rows[0, r, sl] + rows[1, r, sl])
                                          * dv + bvecs[j])

                pltpu.sync_copy(
                    rows.at[2, pl.ds(0, gn)],
                    out_hbm.at[pl.ds(gbase, gn), pl.ds(c * _DH, _DH)])

            assert last <= _K

            @pl.when(s < _NS - 1)
            def _():
                for go, gn in ([(0, _K), (_K, rpw - _K)] if rpw > _K
                               else [(0, rpw)]):
                    fin_group(go, gn)

            @pl.when(s == _NS - 1)
            def _():
                fin_group(0, last)

    return k(hs_split, src, dst, *extra_in)


def _deg_col(parts_ref, nw):
    ones = jnp.ones((nw, 1), jnp.float32)
    d = lax.dot_general(parts_ref[...], ones, (((0,), (0,)), ((), ())),
                        precision=_HIGH, preferred_element_type=jnp.float32)
    return lax.rsqrt(d + 1.0)


def _tc_matmul(x, W1):
    def body(x_ref, w_ref, h_ref):
        h_ref[...] = jnp.dot(x_ref[...], w_ref[...], precision=_HIGH,
                             preferred_element_type=jnp.float32)

    return pl.pallas_call(
        body,
        out_shape=jax.ShapeDtypeStruct((_N1, _D), jnp.float32),
    )(x, W1)


def _tc_scale(h, deg1_parts, deg2_parts):
    def body(h_ref, dp1_ref, dp2_ref, hs_ref, dis1_ref, dis2_ref):
        dis1 = _deg_col(dp1_ref, _NW)
        dis2 = _deg_col(dp2_ref, _NW)
        hs = h_ref[...] * dis1
        hs_ref[0] = hs[:, :_DH]
        hs_ref[1] = hs[:, _DH:]
        dis1_ref[...] = dis1
        dis2_ref[...] = dis2

    return pl.pallas_call(
        body,
        out_shape=[
            jax.ShapeDtypeStruct((_NC, _N1, _DH), jnp.float32),
            jax.ShapeDtypeStruct((_N1, 1), jnp.float32),
            jax.ShapeDtypeStruct((_N2, 1), jnp.float32),
        ],
    )(h, deg1_parts, deg2_parts)


def _tc_mid(acc1_parts, hs1, dis1, b1, agg_W, agg_b, W2, dis2):
    def body(a_ref, hs_ref, dis1_ref, b1_ref, aw_ref, ab_ref, w2_ref, dis2_ref,
             hs2_ref, dis2b_ref):
        full = jnp.concatenate(
            [a_ref[0] + hs_ref[0], a_ref[1] + hs_ref[1]], axis=1)
        h1 = full * dis1_ref[...] + b1_ref[...]
        h1r = h1.reshape(_BATCH, _NUM_NODES, _D)
        dis2r = dis2_ref[...].reshape(_BATCH, _NG, 1)
        for g in range(_NG):
            z = ab_ref[g][None, :] + jnp.zeros((_BATCH, _D), jnp.float32)
            for kk in range(_GS):
                z = z + jnp.dot(h1r[:, g * _GS + kk, :], aw_ref[g, kk],
                                precision=_HIGH,
                                preferred_element_type=jnp.float32)
            g2 = jnp.dot(z, w2_ref[...], precision=_HIGH,
                         preferred_element_type=jnp.float32)
            hs2 = g2 * dis2r[:, g, :]
            hs2_ref[0, :, g, :] = hs2[:, :_DH]
            hs2_ref[1, :, g, :] = hs2[:, _DH:]
            dis2b_ref[:, g, :] = jnp.broadcast_to(dis2r[:, g, :], (_BATCH, 16))

    return pl.pallas_call(
        body,
        out_shape=[
            jax.ShapeDtypeStruct((_NC, _BATCH, _NG, _DH), jnp.float32),
            jax.ShapeDtypeStruct((_BATCH, _NG, 16), jnp.float32),
        ],
    )(acc1_parts, hs1, dis1, b1, agg_W, agg_b, W2, dis2)


def kernel(x, A1, A2, W1, b1, agg_W, agg_b, W2, b2):
    src1, dst1 = A1[0], A1[1]
    src2, dst2 = A2[0], A2[1]

    deg1_parts, deg2_parts = _sc_degrees(dst1, dst2)
    h_raw = _tc_matmul(x, W1)

    hs1s, dis1, dis2 = _tc_scale(h_raw, deg1_parts, deg2_parts)

    acc1_parts = _sc_edge_scatter(hs1s, src1, dst1, _N1, _E1)

    hs2s, dis2b = _tc_mid(
        acc1_parts, hs1s, dis1, b1,
        agg_W.reshape(_NG, _GS, _D, _D), agg_b, W2, dis2,
    )
    hs2s = hs2s.reshape(_NC, _N2, _DH)

    out = _sc_edge_scatter(
        hs2s, src2, dst2, _N2, _E2,
        finalize=(dis2b.reshape(_N2, 16), b2.reshape(_NC, _DH)))
    return out
